# baseline (speedup 1.0000x reference)
import functools
import math

import jax
import jax.numpy as jnp
from jax import lax
from jax.experimental import pallas as pl
from jax.experimental.pallas import tpu as pltpu
from jax.experimental.pallas import tpu_sc as plsc

D_MODEL = 64
SCALE = math.sqrt(D_MODEL)

_NC = 2
_NS = 16
_NW = _NC * _NS
_LANES = 16
_NBUF = 4


def _make_gather_kernel(n_b: int, n_s: int):
    assert n_b % (_NW * _NBUF) == 0 and n_s % 8 == 0
    rows_per_w = n_b // _NW
    n_outer = rows_per_w // _NBUF

    mesh = plsc.VectorSubcoreMesh(core_axis_name="c", subcore_axis_name="s")

    scratch = (
        [pltpu.VMEM((n_s,), jnp.int32) for _ in range(_NBUF)]
        + [pltpu.VMEM((n_s, D_MODEL), jnp.float32) for _ in range(_NBUF)]
        + [pltpu.SemaphoreType.DMA] * (3 * _NBUF)
    )

    @functools.partial(
        pl.kernel,
        out_type=jax.ShapeDtypeStruct((n_b, n_s, D_MODEL), jnp.float32),
        mesh=mesh,
        scratch_types=scratch,
        compiler_params=pltpu.CompilerParams(use_tc_tiling_on_sc=False),
    )
    def gather_scale(lut_hbm, idx_hbm, out_hbm, *sc):
        idx_v = sc[:_NBUF]
        rows_v = sc[_NBUF:2 * _NBUF]
        idx_s = sc[2 * _NBUF:3 * _NBUF]
        in_s = sc[3 * _NBUF:4 * _NBUF]
        out_s = sc[4 * _NBUF:5 * _NBUF]

        wid = lax.axis_index("s") * _NC + lax.axis_index("c")
        base = wid * rows_per_w

        @pl.loop(0, n_outer)
        def _outer(o):
            i0 = base + o * _NBUF

            idx_dma = [
                pltpu.async_copy(
                    idx_hbm.at[pl.ds((i0 + b) * n_s, n_s)],
                    idx_v[b], idx_s[b])
                for b in range(_NBUF)
            ]

            gather_dma = []
            for b in range(_NBUF):
                @pl.when(o > 0)
                def _drain():
                    pltpu.make_async_copy(
                        rows_v[b], out_hbm.at[0], out_s[b]).wait()
                idx_dma[b].wait()
                gather_dma.append(
                    pltpu.async_copy(lut_hbm.at[idx_v[b]], rows_v[b],
                                     in_s[b]))

            for b in range(_NBUF):
                gather_dma[b].wait()

                @pl.loop(0, n_s, unroll=8)
                def _scale(r):
                    for d in range(D_MODEL // _LANES):
                        sl = pl.ds(d * _LANES, _LANES)
                        rows_v[b][r, sl] = rows_v[b][r, sl] * SCALE

                pltpu.async_copy(rows_v[b], out_hbm.at[i0 + b], out_s[b])

        for b in range(_NBUF):
            pltpu.make_async_copy(
                rows_v[b], out_hbm.at[0], out_s[b]).wait()

    return gather_scale


def kernel(x, lut):
    b, s = x.shape
    flat_idx = x.reshape(b * s).astype(jnp.int32)
    return _make_gather_kernel(b, s)(lut, flat_idx)
